# baseline (speedup 1.0000x reference)
import functools

import jax
import jax.numpy as jnp
from jax import lax
from jax.experimental import pallas as pl
from jax.experimental.pallas import tpu as pltpu
from jax.experimental.pallas import tpu_sc as plsc

_D = 64
_V = 1_000_000
_BB = 4096
_S = 50

_info = plsc.get_sparse_core_info()
_NC, _NS = _info.num_cores, _info.num_subcores
_NW = _NC * _NS

_TB = 384
_NPIECE = _V // _TB
_TAIL0 = _NPIECE * _TB
_TAILW = _V - _TAIL0
_KMAX = (_NPIECE + _NW - 1) // _NW
_KPAIR = (_KMAX + 1) // 2

_RPW = _BB // _NW
_CR = 8
_NCHUNK = _RPW // _CR
_NBUF = 4
_LOOK = 2


@functools.partial(
    pl.kernel,
    mesh=plsc.VectorSubcoreMesh(core_axis_name="c", subcore_axis_name="s"),
    out_type=jax.ShapeDtypeStruct((_V * _D,), jnp.float32),
    scratch_types=[
        *[pltpu.VMEM((_D, _TB), jnp.float32) for _ in range(2)],
        *[pltpu.VMEM((_TB * _D,), jnp.float32) for _ in range(2)],
        pltpu.VMEM((_D, _TAILW), jnp.float32),
        *[pltpu.SemaphoreType.DMA for _ in range(4)],
    ],
    compiler_params=pltpu.CompilerParams(use_tc_tiling_on_sc=True,
                                         needs_layout_passes=False),
)
def _transpose_kernel(wt_hbm, tail_hbm, stg_hbm, in0, in1, out0, out1,
                      tail_v, *sems):
    ins = (in0, in1)
    outs = (out0, out1)
    isems = sems[:2]
    osems = sems[2:]
    wid = lax.axis_index("s") * _NC + lax.axis_index("c")
    lanes = lax.iota(jnp.int32, 16)

    def transpose_piece(in_v, out_v):
        def row(c, _):
            for m in range(_D // 16):
                vals = plsc.load_gather(
                    in_v, [16 * m + lanes, jnp.full((16,), c, jnp.int32)])
                out_v[pl.ds(c * _D + 16 * m, 16)] = vals
            return 0

        lax.fori_loop(0, _TB, row, 0)

    def in_start(p, b):
        return pltpu.async_copy(
            wt_hbm.at[:, pl.ds(p * _TB, _TB)], ins[b], isems[b])

    def out_start(p, b):
        return pltpu.async_copy(
            outs[b], stg_hbm.at[pl.ds(p * _TB * _D, _TB * _D)], osems[b])

    def piece_id(k, b):
        return wid + _NW * (2 * k + b)

    for b in range(2):
        @pl.when(piece_id(0, b) < _NPIECE)
        def _(b=b):
            in_start(piece_id(0, b), b)

    def round_body(k, _):
        for b in range(2):
            p = piece_id(k, b)

            @pl.when(p < _NPIECE)
            def _():
                @pl.when(k > 0)
                def _():
                    pltpu.make_async_copy(
                        outs[b],
                        stg_hbm.at[pl.ds(0, _TB * _D)],
                        osems[b]).wait()

                pltpu.make_async_copy(
                    wt_hbm.at[:, pl.ds(0, _TB)], ins[b], isems[b]).wait()
                transpose_piece(ins[b], outs[b])
                out_start(p, b)
                pnext = piece_id(k + 1, b)

                @pl.when(pnext < _NPIECE)
                def _():
                    in_start(pnext, b)
        return 0

    lax.fori_loop(0, _KPAIR, round_body, 0)
    for b in range(2):
        @pl.when(piece_id(0, b) < _NPIECE)
        def _(b=b):
            pltpu.make_async_copy(
                outs[b], stg_hbm.at[pl.ds(0, _TB * _D)], osems[b]).wait()

    @pl.when(wid == 0)
    def _():
        pltpu.sync_copy(tail_hbm, tail_v)

        def row(c, _):
            for m in range(_D // 16):
                vals = plsc.load_gather(
                    tail_v, [16 * m + lanes, jnp.full((16,), c, jnp.int32)])
                out0[pl.ds(c * _D + 16 * m, 16)] = vals
            return 0

        lax.fori_loop(0, _TAILW, row, 0)
        pltpu.sync_copy(out0.at[pl.ds(0, _TAILW * _D)],
                        stg_hbm.at[pl.ds(_TAIL0 * _D, _TAILW * _D)])


@functools.partial(
    pl.kernel,
    mesh=plsc.VectorSubcoreMesh(core_axis_name="c", subcore_axis_name="s"),
    out_type=jax.ShapeDtypeStruct((_BB, _S, _D), jnp.float32),
    scratch_types=[
        pltpu.VMEM((_RPW, _S), jnp.int32),
        *[pltpu.VMEM((_CR, _S, _D), jnp.float32) for _ in range(_NBUF)],
        *[pltpu.SemaphoreType.DMA for _ in range(2 * _NBUF)],
    ],
    compiler_params=pltpu.CompilerParams(use_tc_tiling_on_sc=False),
)
def _gather_kernel(idx_hbm, table_hbm, out_hbm, idx_v, *scratch):
    bufs = scratch[:_NBUF]
    gsems = scratch[_NBUF:2 * _NBUF]
    osems = scratch[2 * _NBUF:]
    wid = lax.axis_index("s") * _NC + lax.axis_index("c")
    base = wid * _RPW
    pltpu.sync_copy(idx_hbm.at[pl.ds(base, _RPW)], idx_v)
    gathers = {}
    outs = {}
    for t in range(_NCHUNK + _LOOK):
        if t < _NCHUNK:
            b = t % _NBUF
            if t >= _NBUF:
                outs[t - _NBUF].wait()
            gathers[t] = [
                pltpu.async_copy(
                    table_hbm.at[idx_v.at[t * _CR + j]],
                    bufs[b].at[j],
                    gsems[b])
                for j in range(_CR)
            ]
        d = t - _LOOK
        if 0 <= d < _NCHUNK:
            for g in gathers[d]:
                g.wait()
            outs[d] = pltpu.async_copy(
                bufs[d % _NBUF], out_hbm.at[pl.ds(base + d * _CR, _CR)],
                osems[d % _NBUF])
    for d in range(_NCHUNK - _NBUF, _NCHUNK):
        outs[d].wait()


def kernel(input, weight):
    wt = weight.T
    stg = _transpose_kernel(wt, wt[:, _TAIL0:])
    return _gather_kernel(input.astype(jnp.int32),
                          stg.reshape(_V, _D))


# speedup vs baseline: 1.2008x; 1.2008x over previous
import functools

import jax
import jax.numpy as jnp
from jax import lax
from jax.experimental import pallas as pl
from jax.experimental.pallas import tpu as pltpu
from jax.experimental.pallas import tpu_sc as plsc

_D = 64
_V = 1_000_000
_BB = 4096
_S = 50

_info = plsc.get_sparse_core_info()
_NC, _NS = _info.num_cores, _info.num_subcores
_NW = _NC * _NS

_TB = 384
_NPIECE = _V // _TB
_TAIL0 = _NPIECE * _TB
_TAILW = _V - _TAIL0
_KMAX = (_NPIECE + _NW - 1) // _NW
_KPAIR = (_KMAX + 1) // 2

_RPW = _BB // _NW
_CR = 8
_NCHUNK = _RPW // _CR
_NBUF = 4
_LOOK = 2


@functools.partial(
    pl.kernel,
    mesh=plsc.VectorSubcoreMesh(core_axis_name="c", subcore_axis_name="s"),
    out_type=jax.ShapeDtypeStruct((_V * _D,), jnp.float32),
    scratch_types=[
        *[pltpu.VMEM((_D, _TB), jnp.float32) for _ in range(2)],
        *[pltpu.VMEM((_TB * _D,), jnp.float32) for _ in range(2)],
        pltpu.VMEM((_D, _TAILW), jnp.float32),
        *[pltpu.SemaphoreType.DMA for _ in range(4)],
    ],
    compiler_params=pltpu.CompilerParams(use_tc_tiling_on_sc=True,
                                         needs_layout_passes=False),
)
def _transpose_kernel(wt_hbm, tail_hbm, stg_hbm, in0, in1, out0, out1,
                      tail_v, *sems):
    ins = (in0, in1)
    outs = (out0, out1)
    isems = sems[:2]
    osems = sems[2:]
    wid = lax.axis_index("s") * _NC + lax.axis_index("c")
    lanes = lax.iota(jnp.int32, 16)
    lanes_d = lanes * _D

    def transpose_piece(in_v, out_v):
        def drow(d, _):
            for cc in range(_TB // 16):
                v = in_v[d, pl.ds(cc * 16, 16)]
                plsc.store_scatter(out_v, [lanes_d + (cc * 16 * _D + d)], v)
            return 0

        lax.fori_loop(0, _D, drow, 0)

    def in_start(p, b):
        return pltpu.async_copy(
            wt_hbm.at[:, pl.ds(p * _TB, _TB)], ins[b], isems[b])

    def out_start(p, b):
        return pltpu.async_copy(
            outs[b], stg_hbm.at[pl.ds(p * _TB * _D, _TB * _D)], osems[b])

    def piece_id(k, b):
        return wid + _NW * (2 * k + b)

    for b in range(2):
        @pl.when(piece_id(0, b) < _NPIECE)
        def _(b=b):
            in_start(piece_id(0, b), b)

    def round_body(k, _):
        for b in range(2):
            p = piece_id(k, b)

            @pl.when(p < _NPIECE)
            def _():
                @pl.when(k > 0)
                def _():
                    pltpu.make_async_copy(
                        outs[b],
                        stg_hbm.at[pl.ds(0, _TB * _D)],
                        osems[b]).wait()

                pltpu.make_async_copy(
                    wt_hbm.at[:, pl.ds(0, _TB)], ins[b], isems[b]).wait()
                transpose_piece(ins[b], outs[b])
                out_start(p, b)
                pnext = piece_id(k + 1, b)

                @pl.when(pnext < _NPIECE)
                def _():
                    in_start(pnext, b)
        return 0

    lax.fori_loop(0, _KPAIR, round_body, 0)
    for b in range(2):
        @pl.when(piece_id(0, b) < _NPIECE)
        def _(b=b):
            pltpu.make_async_copy(
                outs[b], stg_hbm.at[pl.ds(0, _TB * _D)], osems[b]).wait()

    @pl.when(wid == 0)
    def _():
        pltpu.sync_copy(tail_hbm, tail_v)

        def drow(d, _):
            for cc in range(_TAILW // 16):
                v = tail_v[d, pl.ds(cc * 16, 16)]
                plsc.store_scatter(out0, [lanes_d + (cc * 16 * _D + d)], v)
            return 0

        lax.fori_loop(0, _D, drow, 0)
        pltpu.sync_copy(out0.at[pl.ds(0, _TAILW * _D)],
                        stg_hbm.at[pl.ds(_TAIL0 * _D, _TAILW * _D)])


@functools.partial(
    pl.kernel,
    mesh=plsc.VectorSubcoreMesh(core_axis_name="c", subcore_axis_name="s"),
    out_type=jax.ShapeDtypeStruct((_BB, _S, _D), jnp.float32),
    scratch_types=[
        pltpu.VMEM((_RPW, _S), jnp.int32),
        *[pltpu.VMEM((_CR, _S, _D), jnp.float32) for _ in range(_NBUF)],
        *[pltpu.SemaphoreType.DMA for _ in range(2 * _NBUF)],
    ],
    compiler_params=pltpu.CompilerParams(use_tc_tiling_on_sc=False),
)
def _gather_kernel(idx_hbm, table_hbm, out_hbm, idx_v, *scratch):
    bufs = scratch[:_NBUF]
    gsems = scratch[_NBUF:2 * _NBUF]
    osems = scratch[2 * _NBUF:]
    wid = lax.axis_index("s") * _NC + lax.axis_index("c")
    base = wid * _RPW
    pltpu.sync_copy(idx_hbm.at[pl.ds(base, _RPW)], idx_v)
    gathers = {}
    outs = {}
    for t in range(_NCHUNK + _LOOK):
        if t < _NCHUNK:
            b = t % _NBUF
            if t >= _NBUF:
                outs[t - _NBUF].wait()
            gathers[t] = [
                pltpu.async_copy(
                    table_hbm.at[idx_v.at[t * _CR + j]],
                    bufs[b].at[j],
                    gsems[b])
                for j in range(_CR)
            ]
        d = t - _LOOK
        if 0 <= d < _NCHUNK:
            for g in gathers[d]:
                g.wait()
            outs[d] = pltpu.async_copy(
                bufs[d % _NBUF], out_hbm.at[pl.ds(base + d * _CR, _CR)],
                osems[d % _NBUF])
    for d in range(_NCHUNK - _NBUF, _NCHUNK):
        outs[d].wait()


def kernel(input, weight):
    wt = weight.T
    stg = _transpose_kernel(wt, wt[:, _TAIL0:])
    return _gather_kernel(input.astype(jnp.int32),
                          stg.reshape(_V, _D))


# speedup vs baseline: 1.6055x; 1.3371x over previous
import functools

import jax
import jax.numpy as jnp
from jax import lax
from jax.experimental import pallas as pl
from jax.experimental.pallas import tpu as pltpu
from jax.experimental.pallas import tpu_sc as plsc

_D = 64
_V = 1_000_000
_BB = 4096
_S = 50

_info = plsc.get_sparse_core_info()
_NC, _NS = _info.num_cores, _info.num_subcores
_NW = _NC * _NS

_TB = 384
_NPIECE = _V // _TB
_TAIL0 = _NPIECE * _TB
_TAILW = _V - _TAIL0
_KMAX = (_NPIECE + _NW - 1) // _NW
_KPAIR = (_KMAX + 1) // 2

_RPW = _BB // _NW
_CR = 8
_NCHUNK = _RPW // _CR
_NBUF = 4
_LOOK = 2


@functools.partial(
    pl.kernel,
    mesh=plsc.VectorSubcoreMesh(core_axis_name="c", subcore_axis_name="s"),
    out_type=jax.ShapeDtypeStruct((_V * _D,), jnp.float32),
    scratch_types=[
        *[pltpu.VMEM((_D, _TB), jnp.float32) for _ in range(2)],
        *[pltpu.VMEM((_TB * _D,), jnp.float32) for _ in range(2)],
        pltpu.VMEM((_D, _TAILW), jnp.float32),
        *[pltpu.SemaphoreType.DMA for _ in range(4)],
    ],
    compiler_params=pltpu.CompilerParams(use_tc_tiling_on_sc=True,
                                         needs_layout_passes=False),
)
def _transpose_kernel(wt_hbm, tail_hbm, stg_hbm, in0, in1, out0, out1,
                      tail_v, *sems):
    ins = (in0, in1)
    outs = (out0, out1)
    isems = sems[:2]
    osems = sems[2:]
    wid = lax.axis_index("s") * _NC + lax.axis_index("c")
    lanes = lax.iota(jnp.int32, 16)
    lanes_d = lanes * _D

    def transpose_piece(in_v, out_v):
        @plsc.parallel_loop(0, _D, unroll=4)
        def _drow(d):
            for cc in range(_TB // 16):
                v = in_v[d, pl.ds(cc * 16, 16)]
                plsc.store_scatter(out_v, [lanes_d + (cc * 16 * _D + d)], v)

    def in_start(p, b):
        return pltpu.async_copy(
            wt_hbm.at[:, pl.ds(p * _TB, _TB)], ins[b], isems[b])

    def out_start(p, b):
        return pltpu.async_copy(
            outs[b], stg_hbm.at[pl.ds(p * _TB * _D, _TB * _D)], osems[b])

    def piece_id(k, b):
        return wid + _NW * (2 * k + b)

    for b in range(2):
        @pl.when(piece_id(0, b) < _NPIECE)
        def _(b=b):
            in_start(piece_id(0, b), b)

    def round_body(k, _):
        for b in range(2):
            p = piece_id(k, b)

            @pl.when(p < _NPIECE)
            def _():
                @pl.when(k > 0)
                def _():
                    pltpu.make_async_copy(
                        outs[b],
                        stg_hbm.at[pl.ds(0, _TB * _D)],
                        osems[b]).wait()

                pltpu.make_async_copy(
                    wt_hbm.at[:, pl.ds(0, _TB)], ins[b], isems[b]).wait()
                transpose_piece(ins[b], outs[b])
                out_start(p, b)
                pnext = piece_id(k + 1, b)

                @pl.when(pnext < _NPIECE)
                def _():
                    in_start(pnext, b)
        return 0

    lax.fori_loop(0, _KPAIR, round_body, 0)
    for b in range(2):
        @pl.when(piece_id(0, b) < _NPIECE)
        def _(b=b):
            pltpu.make_async_copy(
                outs[b], stg_hbm.at[pl.ds(0, _TB * _D)], osems[b]).wait()

    @pl.when(wid == 0)
    def _():
        pltpu.sync_copy(tail_hbm, tail_v)

        @plsc.parallel_loop(0, _D, unroll=4)
        def _tdrow(d):
            for cc in range(_TAILW // 16):
                v = tail_v[d, pl.ds(cc * 16, 16)]
                plsc.store_scatter(out0, [lanes_d + (cc * 16 * _D + d)], v)
        pltpu.sync_copy(out0.at[pl.ds(0, _TAILW * _D)],
                        stg_hbm.at[pl.ds(_TAIL0 * _D, _TAILW * _D)])


@functools.partial(
    pl.kernel,
    mesh=plsc.VectorSubcoreMesh(core_axis_name="c", subcore_axis_name="s"),
    out_type=jax.ShapeDtypeStruct((_BB, _S, _D), jnp.float32),
    scratch_types=[
        pltpu.VMEM((_RPW, _S), jnp.int32),
        *[pltpu.VMEM((_CR, _S, _D), jnp.float32) for _ in range(_NBUF)],
        *[pltpu.SemaphoreType.DMA for _ in range(2 * _NBUF)],
    ],
    compiler_params=pltpu.CompilerParams(use_tc_tiling_on_sc=False),
)
def _gather_kernel(idx_hbm, table_hbm, out_hbm, idx_v, *scratch):
    bufs = scratch[:_NBUF]
    gsems = scratch[_NBUF:2 * _NBUF]
    osems = scratch[2 * _NBUF:]
    wid = lax.axis_index("s") * _NC + lax.axis_index("c")
    base = wid * _RPW
    pltpu.sync_copy(idx_hbm.at[pl.ds(base, _RPW)], idx_v)
    gathers = {}
    outs = {}
    for t in range(_NCHUNK + _LOOK):
        if t < _NCHUNK:
            b = t % _NBUF
            if t >= _NBUF:
                outs[t - _NBUF].wait()
            gathers[t] = [
                pltpu.async_copy(
                    table_hbm.at[idx_v.at[t * _CR + j]],
                    bufs[b].at[j],
                    gsems[b])
                for j in range(_CR)
            ]
        d = t - _LOOK
        if 0 <= d < _NCHUNK:
            for g in gathers[d]:
                g.wait()
            outs[d] = pltpu.async_copy(
                bufs[d % _NBUF], out_hbm.at[pl.ds(base + d * _CR, _CR)],
                osems[d % _NBUF])
    for d in range(_NCHUNK - _NBUF, _NCHUNK):
        outs[d].wait()


def kernel(input, weight):
    wt = weight.T
    stg = _transpose_kernel(wt, wt[:, _TAIL0:])
    return _gather_kernel(input.astype(jnp.int32),
                          stg.reshape(_V, _D))


# speedup vs baseline: 2.1713x; 1.3524x over previous
import functools

import jax
import jax.numpy as jnp
from jax import lax
from jax.experimental import pallas as pl
from jax.experimental.pallas import tpu as pltpu
from jax.experimental.pallas import tpu_sc as plsc

_D = 64
_BB = 4096
_S = 50

_info = plsc.get_sparse_core_info()
_NC, _NS = _info.num_cores, _info.num_subcores
_NW = _NC * _NS
_RPW = _BB // _NW
_CR = 8
_NCHUNK = _RPW // _CR
_NBUF = 4
_LOOK = 2


@functools.partial(
    pl.kernel,
    mesh=plsc.VectorSubcoreMesh(core_axis_name="c", subcore_axis_name="s"),
    out_type=jax.ShapeDtypeStruct((_BB, _S, _D), jnp.float32),
    scratch_types=[
        pltpu.VMEM((_RPW, _S), jnp.int32),
        *[pltpu.VMEM((_CR, _S, _D), jnp.float32) for _ in range(_NBUF)],
        *[pltpu.SemaphoreType.DMA for _ in range(2 * _NBUF)],
    ],
    compiler_params=pltpu.CompilerParams(use_tc_tiling_on_sc=False),
)
def _gather_kernel(idx_hbm, table_hbm, out_hbm, idx_v, *scratch):
    bufs = scratch[:_NBUF]
    gsems = scratch[_NBUF:2 * _NBUF]
    osems = scratch[2 * _NBUF:]
    wid = lax.axis_index("s") * _NC + lax.axis_index("c")
    base = wid * _RPW
    pltpu.sync_copy(idx_hbm.at[pl.ds(base, _RPW)], idx_v)
    gathers = {}
    outs = {}
    for t in range(_NCHUNK + _LOOK):
        if t < _NCHUNK:
            b = t % _NBUF
            if t >= _NBUF:
                outs[t - _NBUF].wait()
            gathers[t] = [
                pltpu.async_copy(
                    table_hbm.at[idx_v.at[t * _CR + j]],
                    bufs[b].at[j],
                    gsems[b])
                for j in range(_CR)
            ]
        d = t - _LOOK
        if 0 <= d < _NCHUNK:
            for g in gathers[d]:
                g.wait()
            outs[d] = pltpu.async_copy(
                bufs[d % _NBUF], out_hbm.at[pl.ds(base + d * _CR, _CR)],
                osems[d % _NBUF])
    for d in range(_NCHUNK - _NBUF, _NCHUNK):
        outs[d].wait()


def kernel(input, weight):
    return _gather_kernel(input.astype(jnp.int32), weight)
